# Initial kernel scaffold; baseline (speedup 1.0000x reference)
import jax
import jax.numpy as jnp
from jax.experimental import pallas as pl


def kernel(input_ids, token_type_ids, W_word, pos_table, tok_table, gamma, beta):
    raise NotImplementedError("write your pallas kernel here")


# speedup vs baseline: 5.0076x; 5.0076x over previous
import functools

import jax
import jax.numpy as jnp
from jax.experimental import pallas as pl


def _fused_kernel(ids_ref, tt_ref, w_ref, pos_ref, tok_ref, gamma_ref,
                  beta_ref, out_ref):
    ids = ids_ref[0]
    x = jnp.dot(ids, w_ref[...], preferred_element_type=jnp.float32)
    x = x + pos_ref[...]
    tt = tt_ref[0]
    tok = jnp.where(tt == 1, tok_ref[1:2, :], tok_ref[0:1, :])
    x = x + tok
    mean = jnp.mean(x, axis=-1, keepdims=True)
    xc = x - mean
    var = jnp.mean(xc * xc, axis=-1, keepdims=True)
    y = xc * jax.lax.rsqrt(var + 1e-12)
    out_ref[0] = y * gamma_ref[...] + beta_ref[...]


@functools.partial(jax.jit, static_argnames=())
def kernel(input_ids, token_type_ids, W_word, pos_table, tok_table, gamma,
           beta):
    B, S, V = input_ids.shape
    H = W_word.shape[1]
    BS = 512
    grid = (S // BS, B)

    tt3 = token_type_ids.reshape(B, S, 1)
    gamma2 = gamma.reshape(1, H)
    beta2 = beta.reshape(1, H)

    out = pl.pallas_call(
        _fused_kernel,
        grid=grid,
        in_specs=[
            pl.BlockSpec((1, BS, V), lambda i, j: (j, i, 0)),
            pl.BlockSpec((1, BS, 1), lambda i, j: (j, i, 0)),
            pl.BlockSpec((V, H), lambda i, j: (0, 0)),
            pl.BlockSpec((BS, H), lambda i, j: (i, 0)),
            pl.BlockSpec((2, H), lambda i, j: (0, 0)),
            pl.BlockSpec((1, H), lambda i, j: (0, 0)),
            pl.BlockSpec((1, H), lambda i, j: (0, 0)),
        ],
        out_specs=pl.BlockSpec((1, BS, H), lambda i, j: (j, i, 0)),
        out_shape=jax.ShapeDtypeStruct((B, S, H), jnp.float32),
    )(input_ids, tt3, W_word, pos_table, tok_table, gamma2, beta2)
    return out


# speedup vs baseline: 5.8203x; 1.1623x over previous
import functools

import jax
import jax.numpy as jnp
from jax.experimental import pallas as pl


def _fused_kernel(ids_ref, tt_ref, w_ref, pos_ref, tok_ref, gamma_ref,
                  beta_ref, out_ref):
    ids = ids_ref[0]
    x = jnp.dot(ids, w_ref[...], preferred_element_type=jnp.float32)
    x = x + pos_ref[...]
    tt = tt_ref[0]
    tok = jnp.where(tt == 1, tok_ref[1:2, :], tok_ref[0:1, :])
    x = x + tok
    mean = jnp.mean(x, axis=-1, keepdims=True)
    xc = x - mean
    var = jnp.mean(xc * xc, axis=-1, keepdims=True)
    y = xc * jax.lax.rsqrt(var + 1e-12)
    out_ref[0] = y * gamma_ref[...] + beta_ref[...]


@functools.partial(jax.jit, static_argnames=())
def kernel(input_ids, token_type_ids, W_word, pos_table, tok_table, gamma,
           beta):
    B, S, V = input_ids.shape
    H = W_word.shape[1]
    BS = 1024
    grid = (S // BS, B)

    tt3 = token_type_ids.reshape(B, S, 1)
    gamma2 = gamma.reshape(1, H)
    beta2 = beta.reshape(1, H)

    out = pl.pallas_call(
        _fused_kernel,
        grid=grid,
        in_specs=[
            pl.BlockSpec((1, BS, V), lambda i, j: (j, i, 0)),
            pl.BlockSpec((1, BS, 1), lambda i, j: (j, i, 0)),
            pl.BlockSpec((V, H), lambda i, j: (0, 0)),
            pl.BlockSpec((BS, H), lambda i, j: (i, 0)),
            pl.BlockSpec((2, H), lambda i, j: (0, 0)),
            pl.BlockSpec((1, H), lambda i, j: (0, 0)),
            pl.BlockSpec((1, H), lambda i, j: (0, 0)),
        ],
        out_specs=pl.BlockSpec((1, BS, H), lambda i, j: (j, i, 0)),
        out_shape=jax.ShapeDtypeStruct((B, S, H), jnp.float32),
    )(input_ids, tt3, W_word, pos_table, tok_table, gamma2, beta2)
    return out


# speedup vs baseline: 6.2850x; 1.0798x over previous
import functools

import jax
import jax.numpy as jnp
from jax.experimental import pallas as pl


def _fused_kernel(ids_ref, tt_ref, w_ref, pos_ref, tok_ref, gamma_ref,
                  beta_ref, out_ref):
    ids = ids_ref[0]
    x = jnp.dot(ids, w_ref[...], preferred_element_type=jnp.float32)
    x = x + pos_ref[...]
    tt = tt_ref[0]
    tok = jnp.where(tt == 1, tok_ref[1:2, :], tok_ref[0:1, :])
    x = x + tok
    mean = jnp.mean(x, axis=-1, keepdims=True)
    xc = x - mean
    var = jnp.mean(xc * xc, axis=-1, keepdims=True)
    y = xc * jax.lax.rsqrt(var + 1e-12)
    out_ref[0] = y * gamma_ref[...] + beta_ref[...]


@functools.partial(jax.jit, static_argnames=())
def kernel(input_ids, token_type_ids, W_word, pos_table, tok_table, gamma,
           beta):
    B, S, V = input_ids.shape
    H = W_word.shape[1]
    BS = 2048
    grid = (S // BS, B)

    tt3 = token_type_ids.reshape(B, S, 1)
    gamma2 = gamma.reshape(1, H)
    beta2 = beta.reshape(1, H)

    out = pl.pallas_call(
        _fused_kernel,
        grid=grid,
        in_specs=[
            pl.BlockSpec((1, BS, V), lambda i, j: (j, i, 0)),
            pl.BlockSpec((1, BS, 1), lambda i, j: (j, i, 0)),
            pl.BlockSpec((V, H), lambda i, j: (0, 0)),
            pl.BlockSpec((BS, H), lambda i, j: (i, 0)),
            pl.BlockSpec((2, H), lambda i, j: (0, 0)),
            pl.BlockSpec((1, H), lambda i, j: (0, 0)),
            pl.BlockSpec((1, H), lambda i, j: (0, 0)),
        ],
        out_specs=pl.BlockSpec((1, BS, H), lambda i, j: (j, i, 0)),
        out_shape=jax.ShapeDtypeStruct((B, S, H), jnp.float32),
    )(input_ids, tt3, W_word, pos_table, tok_table, gamma2, beta2)
    return out
